# initial kernel scaffold (unmeasured)
import functools

import jax
import jax.numpy as jnp
from jax import lax
from jax.experimental import pallas as pl
from jax.experimental.pallas import tpu as pltpu

B, H, D, BS = 8, 8, 128, 16
NZ = 4
SCALE = D ** -0.5


def kernel(Q, K, V, bt, lens):
    nb_local = K.shape[0]
    nb_slots = bt.shape[1]

    def body(bt_ref, lens_ref, q_ref, k_ref, v_ref, out_ref,
             comm_ref, k_scr, v_scr, copy_sems, send_sems, recv_sems):
        my_x = lax.axis_index("x")
        my_y = lax.axis_index("y")
        my_z = lax.axis_index("z")
        base = my_z * nb_local
        left = (my_z - 1) % NZ
        right = (my_z + 1) % NZ

        barrier = pltpu.get_barrier_semaphore()
        for nbr in (left, right):
            pl.semaphore_signal(
                barrier, inc=1,
                device_id=(my_x, my_y, nbr),
                device_id_type=pl.DeviceIdType.MESH,
            )
        pl.semaphore_wait(barrier, 2)

        for i in range(B):
            q_i = q_ref[i, 0]

            def step(j, carry, i=i, q_i=q_i):
                acc, lacc = carry
                page = bt_ref[i, j]
                rel = page - base
                valid = jnp.logical_and(
                    jnp.logical_and(rel >= 0, rel < nb_local),
                    j < lens_ref[i],
                )
                idx = jnp.clip(rel, 0, nb_local - 1)
                ck = pltpu.make_async_copy(k_ref.at[idx], k_scr, copy_sems.at[0])
                cv = pltpu.make_async_copy(v_ref.at[idx], v_scr, copy_sems.at[1])
                ck.start()
                cv.start()
                ck.wait()
                cv.wait()
                kk = k_scr[...]
                s = jnp.sum(q_i[None, :, :] * kk, axis=2) * SCALE
                p = jnp.exp(s) * jnp.where(valid, 1.0, 0.0)
                pb = p[:, :, None]
                acc = acc + jnp.sum(pb * v_scr[...], axis=0)
                lacc = lacc + jnp.sum(
                    jnp.broadcast_to(pb, (BS, H, D)), axis=0)
                return acc, lacc

            acc, lacc = lax.fori_loop(
                0, nb_slots, step,
                (jnp.zeros((H, D), jnp.float32),
                 jnp.zeros((H, D), jnp.float32)),
            )
            comm_ref[0, i * H:(i + 1) * H, :] = acc
            comm_ref[0, B * H + i * H:B * H + (i + 1) * H, :] = lacc

        for h in range(NZ - 1):
            rdma = pltpu.make_async_remote_copy(
                src_ref=comm_ref.at[h],
                dst_ref=comm_ref.at[h + 1],
                send_sem=send_sems.at[h],
                recv_sem=recv_sems.at[h],
                device_id=(my_x, my_y, right),
                device_id_type=pl.DeviceIdType.MESH,
            )
            rdma.start()
            rdma.wait()

        tot = comm_ref[0] + comm_ref[1] + comm_ref[2] + comm_ref[3]
        res = tot[:B * H, :] / tot[B * H:, :]
        for i in range(B):
            out_ref[i, 0, :, :] = res[i * H:(i + 1) * H, :]

        @functools.partial(
            pl.run_scoped, second_barrier=pltpu.SemaphoreType.REGULAR)
        def _(second_barrier):
            for nbr in (left, right):
                pl.semaphore_signal(
                    second_barrier, inc=1,
                    device_id=(my_x, my_y, nbr),
                    device_id_type=pl.DeviceIdType.MESH,
                )
            pl.semaphore_wait(second_barrier, 2)

    out_shape = jax.ShapeDtypeStruct((B, 1, H, D), jnp.float32)
    return pl.pallas_call(
        body,
        out_shape=out_shape,
        in_specs=[
            pl.BlockSpec(memory_space=pltpu.SMEM),
            pl.BlockSpec(memory_space=pltpu.SMEM),
            pl.BlockSpec(memory_space=pltpu.VMEM),
            pl.BlockSpec(memory_space=pltpu.ANY),
            pl.BlockSpec(memory_space=pltpu.ANY),
        ],
        out_specs=pl.BlockSpec(memory_space=pltpu.VMEM),
        scratch_shapes=[
            pltpu.VMEM((NZ, 2 * B * H, D), jnp.float32),
            pltpu.VMEM((BS, H, D), jnp.float32),
            pltpu.VMEM((BS, H, D), jnp.float32),
            pltpu.SemaphoreType.DMA((2,)),
            pltpu.SemaphoreType.DMA((NZ - 1,)),
            pltpu.SemaphoreType.DMA((NZ - 1,)),
        ],
        compiler_params=pltpu.CompilerParams(collective_id=0),
    )(bt, lens, Q, K, V)


# baseline (device time: 4292980 ns/iter reference)
import functools

import jax
import jax.numpy as jnp
from jax import lax
from jax.experimental import pallas as pl
from jax.experimental.pallas import tpu as pltpu

B, H, D, BS = 8, 8, 128, 16
NZ = 4
SCALE = D ** -0.5


def kernel(Q, K, V, bt, lens):
    nb_local = K.shape[0]
    nb_slots = bt.shape[1]

    def body(bt_ref, lens_ref, q_ref, k_ref, v_ref, out_ref,
             comm_ref, k_scr, v_scr, copy_sems, send_sems, recv_sems):
        my_x = lax.axis_index("x")
        my_y = lax.axis_index("y")
        my_z = lax.axis_index("z")
        base = my_z * nb_local
        left = (my_z - 1) % NZ
        right = (my_z + 1) % NZ

        barrier = pltpu.get_barrier_semaphore()
        for nbr in (left, right):
            pl.semaphore_signal(
                barrier, inc=1,
                device_id=(my_x, my_y, nbr),
                device_id_type=pl.DeviceIdType.MESH,
            )
        pl.semaphore_wait(barrier, 2)

        for i in range(B):
            q_i = q_ref[i, 0]

            def step(j, carry, i=i, q_i=q_i):
                acc, lacc = carry
                page = bt_ref[i, j]
                rel = page - base
                valid = jnp.logical_and(
                    jnp.logical_and(rel >= 0, rel < nb_local),
                    j < lens_ref[i],
                )
                idx = jnp.clip(rel, 0, nb_local - 1)
                ck = pltpu.make_async_copy(k_ref.at[idx], k_scr, copy_sems.at[0])
                cv = pltpu.make_async_copy(v_ref.at[idx], v_scr, copy_sems.at[1])
                ck.start()
                cv.start()
                ck.wait()
                cv.wait()
                kk = k_scr[...]
                s = jnp.sum(q_i[None, :, :] * kk, axis=2) * SCALE
                p = jnp.exp(s) * jnp.where(valid, 1.0, 0.0)
                pb = p[:, :, None]
                acc = acc + jnp.sum(pb * v_scr[...], axis=0)
                lacc = lacc + jnp.sum(
                    jnp.broadcast_to(pb, (BS, H, D)), axis=0)
                return acc, lacc

            acc, lacc = lax.fori_loop(
                0, nb_slots, step,
                (jnp.zeros((H, D), jnp.float32),
                 jnp.zeros((H, D), jnp.float32)),
            )
            comm_ref[0, i * H:(i + 1) * H, :] = acc
            comm_ref[0, B * H + i * H:B * H + (i + 1) * H, :] = lacc

        for h in range(NZ - 1):
            rdma = pltpu.make_async_remote_copy(
                src_ref=comm_ref.at[h],
                dst_ref=comm_ref.at[h + 1],
                send_sem=send_sems.at[h],
                recv_sem=recv_sems.at[h],
                device_id=(my_x, my_y, right),
                device_id_type=pl.DeviceIdType.MESH,
            )
            rdma.start()
            rdma.wait()

        tot = comm_ref[0] + comm_ref[1] + comm_ref[2] + comm_ref[3]
        res = tot[:B * H, :] / tot[B * H:, :]
        for i in range(B):
            out_ref[i, 0, :, :] = res[i * H:(i + 1) * H, :]

        @functools.partial(
            pl.run_scoped, second_barrier=pltpu.SemaphoreType.REGULAR)
        def _(second_barrier):
            for nbr in (left, right):
                pl.semaphore_signal(
                    second_barrier, inc=1,
                    device_id=(my_x, my_y, nbr),
                    device_id_type=pl.DeviceIdType.MESH,
                )
            pl.semaphore_wait(second_barrier, 2)

    out_shape = jax.ShapeDtypeStruct((B, 1, H, D), jnp.float32)
    return pl.pallas_call(
        body,
        out_shape=out_shape,
        in_specs=[
            pl.BlockSpec(memory_space=pltpu.SMEM),
            pl.BlockSpec(memory_space=pltpu.SMEM),
            pl.BlockSpec(memory_space=pltpu.VMEM),
            pl.BlockSpec(memory_space=pl.ANY),
            pl.BlockSpec(memory_space=pl.ANY),
        ],
        out_specs=pl.BlockSpec(memory_space=pltpu.VMEM),
        scratch_shapes=[
            pltpu.VMEM((NZ, 2 * B * H, D), jnp.float32),
            pltpu.VMEM((BS, H, D), jnp.float32),
            pltpu.VMEM((BS, H, D), jnp.float32),
            pltpu.SemaphoreType.DMA((2,)),
            pltpu.SemaphoreType.DMA((NZ - 1,)),
            pltpu.SemaphoreType.DMA((NZ - 1,)),
        ],
        compiler_params=pltpu.CompilerParams(collective_id=0),
    )(bt, lens, Q, K, V)


# device time: 211449 ns/iter; 20.3027x vs baseline; 20.3027x over previous
import functools

import jax
import jax.numpy as jnp
from jax import lax
from jax.experimental import pallas as pl
from jax.experimental.pallas import tpu as pltpu

B, H, D, BS = 8, 8, 128, 16
NZ = 4
SCALE = D ** -0.5


def kernel(Q, K, V, bt, lens):
    nb_local = K.shape[0]
    nb_slots = bt.shape[1]

    NBUF = 4
    DEPTH = 3

    def body(bt_ref, lens_ref, q_ref, k_ref, v_ref, out_ref,
             comm_ref, idx_ref, k_scr, v_scr, sem_k, sem_v,
             send_sems, recv_sems):
        my_x = lax.axis_index("x")
        my_y = lax.axis_index("y")
        my_z = lax.axis_index("z")
        base = my_z * nb_local
        left = (my_z - 1) % NZ
        right = (my_z + 1) % NZ

        barrier = pltpu.get_barrier_semaphore()
        for nbr in (left, right):
            pl.semaphore_signal(
                barrier, inc=1,
                device_id=(my_x, my_y, nbr),
                device_id_type=pl.DeviceIdType.MESH,
            )
        pl.semaphore_wait(barrier, 2)

        for i in range(B):
            q_i = q_ref[i, 0]

            def scan(j, cnt, i=i):
                page = bt_ref[i, j]
                rel = page - base
                valid = jnp.logical_and(
                    jnp.logical_and(rel >= 0, rel < nb_local),
                    j < lens_ref[i],
                )

                @pl.when(valid)
                def _():
                    idx_ref[cnt] = rel

                return cnt + jnp.where(valid, 1, 0)

            n = lax.fori_loop(0, nb_slots, scan, jnp.int32(0))

            def start_fetch(t):
                slot = lax.rem(t, NBUF)
                idx = idx_ref[t]
                pltpu.make_async_copy(
                    k_ref.at[idx], k_scr.at[slot], sem_k.at[slot]).start()
                pltpu.make_async_copy(
                    v_ref.at[idx], v_scr.at[slot], sem_v.at[slot]).start()

            for w in range(DEPTH):
                @pl.when(w < n)
                def _(w=w):
                    start_fetch(jnp.int32(w))

            def step(t, carry, q_i=q_i):
                acc, lacc = carry
                slot = lax.rem(t, NBUF)

                @pl.when(t + DEPTH < n)
                def _():
                    start_fetch(t + DEPTH)

                pltpu.make_async_copy(
                    k_ref.at[0], k_scr.at[slot], sem_k.at[slot]).wait()
                pltpu.make_async_copy(
                    v_ref.at[0], v_scr.at[slot], sem_v.at[slot]).wait()
                kk = k_scr[slot]
                s = jnp.sum(q_i[None, :, :] * kk, axis=2) * SCALE
                p = jnp.exp(s)
                pb = p[:, :, None]
                acc = acc + jnp.sum(pb * v_scr[slot], axis=0)
                lacc = lacc + jnp.sum(
                    jnp.broadcast_to(pb, (BS, H, D)), axis=0)
                return acc, lacc

            acc, lacc = lax.fori_loop(
                0, n, step,
                (jnp.zeros((H, D), jnp.float32),
                 jnp.zeros((H, D), jnp.float32)),
            )
            comm_ref[0, i * H:(i + 1) * H, :] = acc
            comm_ref[0, B * H + i * H:B * H + (i + 1) * H, :] = lacc

        for h in range(NZ - 1):
            rdma = pltpu.make_async_remote_copy(
                src_ref=comm_ref.at[h],
                dst_ref=comm_ref.at[h + 1],
                send_sem=send_sems.at[h],
                recv_sem=recv_sems.at[h],
                device_id=(my_x, my_y, right),
                device_id_type=pl.DeviceIdType.MESH,
            )
            rdma.start()
            rdma.wait()

        tot = comm_ref[0] + comm_ref[1] + comm_ref[2] + comm_ref[3]
        res = tot[:B * H, :] / tot[B * H:, :]
        for i in range(B):
            out_ref[i, 0, :, :] = res[i * H:(i + 1) * H, :]

        @functools.partial(
            pl.run_scoped, second_barrier=pltpu.SemaphoreType.REGULAR)
        def _(second_barrier):
            for nbr in (left, right):
                pl.semaphore_signal(
                    second_barrier, inc=1,
                    device_id=(my_x, my_y, nbr),
                    device_id_type=pl.DeviceIdType.MESH,
                )
            pl.semaphore_wait(second_barrier, 2)

    out_shape = jax.ShapeDtypeStruct((B, 1, H, D), jnp.float32)
    return pl.pallas_call(
        body,
        out_shape=out_shape,
        in_specs=[
            pl.BlockSpec(memory_space=pltpu.SMEM),
            pl.BlockSpec(memory_space=pltpu.SMEM),
            pl.BlockSpec(memory_space=pltpu.VMEM),
            pl.BlockSpec(memory_space=pl.ANY),
            pl.BlockSpec(memory_space=pl.ANY),
        ],
        out_specs=pl.BlockSpec(memory_space=pltpu.VMEM),
        scratch_shapes=[
            pltpu.VMEM((NZ, 2 * B * H, D), jnp.float32),
            pltpu.SMEM((nb_slots,), jnp.int32),
            pltpu.VMEM((NBUF, BS, H, D), jnp.float32),
            pltpu.VMEM((NBUF, BS, H, D), jnp.float32),
            pltpu.SemaphoreType.DMA((NBUF,)),
            pltpu.SemaphoreType.DMA((NBUF,)),
            pltpu.SemaphoreType.DMA((NZ - 1,)),
            pltpu.SemaphoreType.DMA((NZ - 1,)),
        ],
        compiler_params=pltpu.CompilerParams(collective_id=0),
    )(bt, lens, Q, K, V)


# device time: 89824 ns/iter; 47.7932x vs baseline; 2.3540x over previous
import functools

import jax
import jax.numpy as jnp
from jax import lax
from jax.experimental import pallas as pl
from jax.experimental.pallas import tpu as pltpu

B, H, D, BS = 8, 8, 128, 16
NZ = 4
SCALE = D ** -0.5


def kernel(Q, K, V, bt, lens):
    nb_local = K.shape[0]
    nb_slots = bt.shape[1]

    NBUF = 3
    DEPTH = 2
    G = 8

    def body(bt_ref, lens_ref, q_ref, k_ref, v_ref, out_ref,
             comm_ref, idx_ref, k_scr, v_scr, sem_k, sem_v,
             send_sems, recv_sems):
        my_x = lax.axis_index("x")
        my_y = lax.axis_index("y")
        my_z = lax.axis_index("z")
        base = my_z * nb_local
        left = (my_z - 1) % NZ
        right = (my_z + 1) % NZ

        barrier = pltpu.get_barrier_semaphore()
        for nbr in (left, right):
            pl.semaphore_signal(
                barrier, inc=1,
                device_id=(my_x, my_y, nbr),
                device_id_type=pl.DeviceIdType.MESH,
            )
        pl.semaphore_wait(barrier, 2)

        for i in range(B):
            q_i = q_ref[i, 0]

            def scan(j, cnt, i=i):
                rel = bt_ref[i, j] - base
                valid = jnp.logical_and(rel >= 0, rel < nb_local)

                @pl.when(valid)
                def _():
                    idx_ref[cnt] = rel

                return cnt + jnp.where(valid, 1, 0)

            n = lax.fori_loop(0, lens_ref[i], scan, jnp.int32(0))
            nm1 = jnp.maximum(n - 1, 0)
            ngroups = (n + G - 1) // G

            def start_group(g):
                slot = lax.rem(g, NBUF)
                for u in range(G):
                    idx = idx_ref[jnp.minimum(g * G + u, nm1)]
                    pltpu.make_async_copy(
                        k_ref.at[idx],
                        k_scr.at[slot, pl.ds(u * BS, BS)],
                        sem_k.at[slot, u]).start()
                    pltpu.make_async_copy(
                        v_ref.at[idx],
                        v_scr.at[slot, pl.ds(u * BS, BS)],
                        sem_v.at[slot, u]).start()

            for w in range(DEPTH):
                @pl.when(w < ngroups)
                def _(w=w):
                    start_group(jnp.int32(w))

            upage = jax.lax.broadcasted_iota(jnp.int32, (G * BS, H), 0) // BS

            def step(g, carry, q_i=q_i):
                acc, lacc = carry
                slot = lax.rem(g, NBUF)

                @pl.when(g + DEPTH < ngroups)
                def _():
                    start_group(g + DEPTH)

                for u in range(G):
                    pltpu.make_async_copy(
                        k_ref.at[0], k_scr.at[slot, pl.ds(u * BS, BS)],
                        sem_k.at[slot, u]).wait()
                    pltpu.make_async_copy(
                        v_ref.at[0], v_scr.at[slot, pl.ds(u * BS, BS)],
                        sem_v.at[slot, u]).wait()
                kk = k_scr[slot]
                s = jnp.sum(q_i[None, :, :] * kk, axis=2) * SCALE
                mask = (g * G + upage) < n
                p = jnp.exp(s) * mask.astype(jnp.float32)
                pb = p[:, :, None]
                acc = acc + jnp.sum(pb * v_scr[slot], axis=0)
                lacc = lacc + jnp.sum(
                    jnp.broadcast_to(pb, (G * BS, H, D)), axis=0)
                return acc, lacc

            acc, lacc = lax.fori_loop(
                0, ngroups, step,
                (jnp.zeros((H, D), jnp.float32),
                 jnp.zeros((H, D), jnp.float32)),
            )
            comm_ref[0, i * H:(i + 1) * H, :] = acc
            comm_ref[0, B * H + i * H:B * H + (i + 1) * H, :] = lacc

        for h in range(NZ - 1):
            rdma = pltpu.make_async_remote_copy(
                src_ref=comm_ref.at[h],
                dst_ref=comm_ref.at[h + 1],
                send_sem=send_sems.at[h],
                recv_sem=recv_sems.at[h],
                device_id=(my_x, my_y, right),
                device_id_type=pl.DeviceIdType.MESH,
            )
            rdma.start()
            rdma.wait()

        tot = comm_ref[0] + comm_ref[1] + comm_ref[2] + comm_ref[3]
        res = tot[:B * H, :] / tot[B * H:, :]
        for i in range(B):
            out_ref[i, 0, :, :] = res[i * H:(i + 1) * H, :]

        @functools.partial(
            pl.run_scoped, second_barrier=pltpu.SemaphoreType.REGULAR)
        def _(second_barrier):
            for nbr in (left, right):
                pl.semaphore_signal(
                    second_barrier, inc=1,
                    device_id=(my_x, my_y, nbr),
                    device_id_type=pl.DeviceIdType.MESH,
                )
            pl.semaphore_wait(second_barrier, 2)

    out_shape = jax.ShapeDtypeStruct((B, 1, H, D), jnp.float32)
    return pl.pallas_call(
        body,
        out_shape=out_shape,
        in_specs=[
            pl.BlockSpec(memory_space=pltpu.SMEM),
            pl.BlockSpec(memory_space=pltpu.SMEM),
            pl.BlockSpec(memory_space=pltpu.VMEM),
            pl.BlockSpec(memory_space=pl.ANY),
            pl.BlockSpec(memory_space=pl.ANY),
        ],
        out_specs=pl.BlockSpec(memory_space=pltpu.VMEM),
        scratch_shapes=[
            pltpu.VMEM((NZ, 2 * B * H, D), jnp.float32),
            pltpu.SMEM((nb_slots,), jnp.int32),
            pltpu.VMEM((NBUF, G * BS, H, D), jnp.float32),
            pltpu.VMEM((NBUF, G * BS, H, D), jnp.float32),
            pltpu.SemaphoreType.DMA((NBUF, G)),
            pltpu.SemaphoreType.DMA((NBUF, G)),
            pltpu.SemaphoreType.DMA((NZ - 1,)),
            pltpu.SemaphoreType.DMA((NZ - 1,)),
        ],
        compiler_params=pltpu.CompilerParams(collective_id=0),
    )(bt, lens, Q, K, V)


# device time: 80417 ns/iter; 53.3840x vs baseline; 1.1170x over previous
import jax
import jax.numpy as jnp
from jax import lax
from jax.experimental import pallas as pl
from jax.experimental.pallas import tpu as pltpu

B, H, D, BS = 8, 8, 128, 16
NZ = 4
SCALE = D ** -0.5


def kernel(Q, K, V, bt, lens):
    nb_local = K.shape[0]
    nb_slots = bt.shape[1]

    NBUF = 4
    DEPTH = 3
    G = 8

    def body(bt_ref, lens_ref, q_ref, k_ref, v_ref, out_ref,
             comm_ref, idx_ref, k_scr, v_scr, sem_k, sem_v,
             send_sems, recv_sems):
        my_x = lax.axis_index("x")
        my_y = lax.axis_index("y")
        my_z = lax.axis_index("z")
        base = my_z * nb_local
        peers = [(my_z + d) % NZ for d in (1, 2, 3)]

        barrier = pltpu.get_barrier_semaphore()
        for p in peers:
            pl.semaphore_signal(
                barrier, inc=1,
                device_id=(my_x, my_y, p),
                device_id_type=pl.DeviceIdType.MESH,
            )
        pl.semaphore_wait(barrier, 3)

        def peer_rdma(i, d):
            return pltpu.make_async_remote_copy(
                src_ref=comm_ref.at[0, i],
                dst_ref=comm_ref.at[d, i],
                send_sem=send_sems.at[i, d - 1],
                recv_sem=recv_sems.at[i, d - 1],
                device_id=(my_x, my_y, peers[d - 1]),
                device_id_type=pl.DeviceIdType.MESH,
            )

        for i in range(B):
            q_i = q_ref[i, 0]

            def scan(j, cnt, i=i):
                rel = bt_ref[i, j] - base
                valid = jnp.logical_and(rel >= 0, rel < nb_local)

                @pl.when(valid)
                def _():
                    idx_ref[cnt] = rel

                return cnt + jnp.where(valid, 1, 0)

            n = lax.fori_loop(0, lens_ref[i], scan, jnp.int32(0))
            nm1 = jnp.maximum(n - 1, 0)
            ngroups = (n + G - 1) // G

            def start_group(g):
                slot = lax.rem(g, NBUF)
                for u in range(G):
                    idx = idx_ref[jnp.minimum(g * G + u, nm1)]
                    pltpu.make_async_copy(
                        k_ref.at[idx],
                        k_scr.at[slot, pl.ds(u * BS, BS)],
                        sem_k.at[slot, u]).start()
                    pltpu.make_async_copy(
                        v_ref.at[idx],
                        v_scr.at[slot, pl.ds(u * BS, BS)],
                        sem_v.at[slot, u]).start()

            for w in range(DEPTH):
                @pl.when(w < ngroups)
                def _(w=w):
                    start_group(jnp.int32(w))

            upage = jax.lax.broadcasted_iota(jnp.int32, (G * BS, H), 0) // BS

            def step(g, carry, q_i=q_i):
                acc, lacc = carry
                slot = lax.rem(g, NBUF)

                @pl.when(g + DEPTH < ngroups)
                def _():
                    start_group(g + DEPTH)

                for u in range(G):
                    pltpu.make_async_copy(
                        k_ref.at[0], k_scr.at[slot, pl.ds(u * BS, BS)],
                        sem_k.at[slot, u]).wait()
                    pltpu.make_async_copy(
                        v_ref.at[0], v_scr.at[slot, pl.ds(u * BS, BS)],
                        sem_v.at[slot, u]).wait()
                kk = k_scr[slot]
                s = jnp.sum(q_i[None, :, :] * kk, axis=2) * SCALE
                mask = (g * G + upage) < n
                p = jnp.exp(s) * mask.astype(jnp.float32)
                pb = p[:, :, None]
                acc = acc + jnp.sum(pb * v_scr[slot], axis=0)
                lacc = lacc + jnp.sum(
                    jnp.broadcast_to(pb, (G * BS, H, D)), axis=0)
                return acc, lacc

            acc, lacc = lax.fori_loop(
                0, ngroups, step,
                (jnp.zeros((H, D), jnp.float32),
                 jnp.zeros((H, D), jnp.float32)),
            )
            comm_ref[0, i, :H, :] = acc
            comm_ref[0, i, H:, :] = lacc

            for d in (1, 2, 3):
                peer_rdma(i, d).start()

        for i in range(B):
            for d in (1, 2, 3):
                r = peer_rdma(i, d)
                r.wait_send()
                r.wait_recv()

        tot = (comm_ref[0] + comm_ref[1]
               + comm_ref[2] + comm_ref[3])
        out_ref[:, 0, :, :] = tot[:, :H, :] / tot[:, H:, :]


    out_shape = jax.ShapeDtypeStruct((B, 1, H, D), jnp.float32)
    return pl.pallas_call(
        body,
        out_shape=out_shape,
        in_specs=[
            pl.BlockSpec(memory_space=pltpu.SMEM),
            pl.BlockSpec(memory_space=pltpu.SMEM),
            pl.BlockSpec(memory_space=pltpu.VMEM),
            pl.BlockSpec(memory_space=pl.ANY),
            pl.BlockSpec(memory_space=pl.ANY),
        ],
        out_specs=pl.BlockSpec(memory_space=pltpu.VMEM),
        scratch_shapes=[
            pltpu.VMEM((NZ, B, 2 * H, D), jnp.float32),
            pltpu.SMEM((nb_slots,), jnp.int32),
            pltpu.VMEM((NBUF, G * BS, H, D), jnp.float32),
            pltpu.VMEM((NBUF, G * BS, H, D), jnp.float32),
            pltpu.SemaphoreType.DMA((NBUF, G)),
            pltpu.SemaphoreType.DMA((NBUF, G)),
            pltpu.SemaphoreType.DMA((B, 3)),
            pltpu.SemaphoreType.DMA((B, 3)),
        ],
        compiler_params=pltpu.CompilerParams(collective_id=0),
    )(bt, lens, Q, K, V)


# device time: 59363 ns/iter; 72.3174x vs baseline; 1.3547x over previous
import jax
import jax.numpy as jnp
from jax import lax
from jax.experimental import pallas as pl
from jax.experimental.pallas import tpu as pltpu

B, H, D, BS = 8, 8, 128, 16
NZ = 4
SCALE = D ** -0.5


def kernel(Q, K, V, bt, lens):
    nb_local = K.shape[0]
    nb_slots = bt.shape[1]

    NBUF = 6
    DEPTH = 5
    G = 32
    MAXG = B * (nb_slots // G + 1)

    def body(bt_ref, lens_ref, q_ref, k_ref, v_ref, out_ref,
             comm_ref, idx_ref, gbatch_ref, gcount_ref, k_scr, v_scr,
             sem_k, sem_v, send_sems, recv_sems):
        my_x = lax.axis_index("x")
        my_y = lax.axis_index("y")
        my_z = lax.axis_index("z")
        base = my_z * nb_local
        peers = [(my_z + d) % NZ for d in (1, 2, 3)]

        barrier = pltpu.get_barrier_semaphore()
        for p in peers:
            pl.semaphore_signal(
                barrier, inc=1,
                device_id=(my_x, my_y, p),
                device_id_type=pl.DeviceIdType.MESH,
            )
        pl.semaphore_wait(barrier, 3)

        def peer_rdma(i, d):
            return pltpu.make_async_remote_copy(
                src_ref=comm_ref.at[0, i],
                dst_ref=comm_ref.at[d, i],
                send_sem=send_sems.at[i, d - 1],
                recv_sem=recv_sems.at[i, d - 1],
                device_id=(my_x, my_y, peers[d - 1]),
                device_id_type=pl.DeviceIdType.MESH,
            )

        gcnt = jnp.int32(0)
        ng_per_batch = []
        for i in range(B):
            toff = gcnt * G

            def emit(j, cnt, i=i):
                rel = bt_ref[i, j] - base
                v = jnp.logical_and(rel >= 0, rel < nb_local)

                @pl.when(v)
                def _():
                    idx_ref[cnt] = rel

                return cnt + jnp.where(v, 1, 0)

            def scan4(jj, cnt, i=i):
                for u in range(4):
                    cnt = emit(4 * jj + u, cnt)
                return cnt

            li = lens_ref[i]
            cnt = lax.fori_loop(0, li // 4, scan4, toff)
            cnt = lax.fori_loop((li // 4) * 4, li, emit, cnt)
            n = cnt - toff
            for w in range(G - 1):
                idx_ref[cnt + w] = idx_ref[toff]
            ng = (n + G - 1) // G

            def meta(k, _, i=i, gcnt=gcnt, n=n):
                gbatch_ref[gcnt + k] = jnp.int32(i)
                gcount_ref[gcnt + k] = jnp.minimum(n - k * G, G)
                return 0

            lax.fori_loop(0, ng, meta, 0)
            ng_per_batch.append(ng)
            gcnt = gcnt + ng
        gbatch_ref[gcnt] = jnp.int32(B)

        PR = BS * H

        def start_group(g):
            slot = lax.rem(g, NBUF)
            for u in range(G):
                idx = idx_ref[g * G + u]
                pltpu.make_async_copy(
                    k_ref.at[idx],
                    k_scr.at[slot, pl.ds(u * PR, PR)],
                    sem_k.at[slot]).start()
                pltpu.make_async_copy(
                    v_ref.at[idx],
                    v_scr.at[slot, pl.ds(u * PR, PR)],
                    sem_v.at[slot]).start()

        for w in range(DEPTH):
            @pl.when(w < gcnt)
            def _(w=w):
                start_group(jnp.int32(w))

        rows = G * PR
        onehot = (jax.lax.broadcasted_iota(jnp.int32, (rows, H), 0) % H
                  == jax.lax.broadcasted_iota(jnp.int32, (rows, H), 1)
                  ).astype(jnp.float32)
        upage = jax.lax.broadcasted_iota(jnp.int32, (rows, H), 0) // PR

        def step(g, carry):
            acc, lrow = carry
            slot = lax.rem(g, NBUF)

            @pl.when(g + DEPTH < gcnt)
            def _():
                start_group(g + DEPTH)

            pltpu.make_async_copy(
                k_scr.at[slot], k_scr.at[slot], sem_k.at[slot]).wait()
            pltpu.make_async_copy(
                v_scr.at[slot], v_scr.at[slot], sem_v.at[slot]).wait()

            b = gbatch_ref[g]
            m = gcount_ref[g]
            q_b = q_ref[b, 0]
            kk = k_scr[slot]
            s_all = lax.dot_general(
                kk, q_b, (((1,), (1,)), ((), ())),
                preferred_element_type=jnp.float32) * SCALE
            e = (jnp.exp(s_all) * onehot
                 * (upage < m).astype(jnp.float32))
            acc = acc + lax.dot_general(
                e, v_scr[slot], (((0,), (0,)), ((), ())),
                preferred_element_type=jnp.float32)
            lrow = lrow + jnp.sum(e, axis=0, keepdims=True)

            last = gbatch_ref[g + 1] != b

            @pl.when(last)
            def _():
                comm_ref[0, b, :H, :] = acc
                comm_ref[0, b, H:, :] = jnp.broadcast_to(lrow.T, (H, D))
                for d in (1, 2, 3):
                    peer_rdma(b, d).start()

            zero = jnp.where(last, 0.0, 1.0)
            return acc * zero, lrow * zero

        lax.fori_loop(
            0, gcnt, step,
            (jnp.zeros((H, D), jnp.float32), jnp.zeros((1, H), jnp.float32)))

        for i in range(B):
            @pl.when(ng_per_batch[i] == 0)
            def _(i=i):
                comm_ref[0, i] = jnp.zeros((2 * H, D), jnp.float32)
                for d in (1, 2, 3):
                    peer_rdma(i, d).start()

        for i in range(B):
            for d in (1, 2, 3):
                r = peer_rdma(i, d)
                r.wait_send()
                r.wait_recv()

        tot = (comm_ref[0] + comm_ref[1]
               + comm_ref[2] + comm_ref[3])
        out_ref[:, 0, :, :] = tot[:, :H, :] / tot[:, H:, :]


    out_shape = jax.ShapeDtypeStruct((B, 1, H, D), jnp.float32)
    return pl.pallas_call(
        body,
        out_shape=out_shape,
        in_specs=[
            pl.BlockSpec(memory_space=pltpu.SMEM),
            pl.BlockSpec(memory_space=pltpu.SMEM),
            pl.BlockSpec(memory_space=pltpu.VMEM),
            pl.BlockSpec(memory_space=pl.ANY),
            pl.BlockSpec(memory_space=pl.ANY),
        ],
        out_specs=pl.BlockSpec(memory_space=pltpu.VMEM),
        scratch_shapes=[
            pltpu.VMEM((NZ, B, 2 * H, D), jnp.float32),
            pltpu.SMEM((B * (nb_slots + G),), jnp.int32),
            pltpu.SMEM((MAXG + 1,), jnp.int32),
            pltpu.SMEM((MAXG,), jnp.int32),
            pltpu.VMEM((NBUF, G * BS * H, D), jnp.float32),
            pltpu.VMEM((NBUF, G * BS * H, D), jnp.float32),
            pltpu.SemaphoreType.DMA((NBUF,)),
            pltpu.SemaphoreType.DMA((NBUF,)),
            pltpu.SemaphoreType.DMA((B, 3)),
            pltpu.SemaphoreType.DMA((B, 3)),
        ],
        compiler_params=pltpu.CompilerParams(collective_id=0),
    )(bt, lens, Q,
      K.reshape(nb_local, BS * H, D),
      V.reshape(nb_local, BS * H, D))
